# baseline (speedup 1.0000x reference)
import jax
import jax.numpy as jnp
from jax import lax
from jax.experimental import pallas as pl
from jax.experimental.pallas import tpu as pltpu

E = 8
TOP_K = 2
H = 1024
FFN = 512
SFFN = 2048
T = 2048
TM = 256
FJ = 512


def _routed_body(x_ref, rw_ref, gu_ref, dn_ref, out_ref, acc_ref):
    e = pl.program_id(0)
    t = pl.program_id(1)
    sl = pl.ds(t * TM, TM)
    x = x_ref[sl, :]

    logits = lax.dot_general(x, rw_ref[...], (((1,), (1,)), ((), ())),
                             preferred_element_type=jnp.float32)
    col = lax.broadcasted_iota(jnp.int32, (TM, E), 1)
    m1 = jnp.max(logits, axis=1, keepdims=True)
    a1 = jnp.min(jnp.where(logits == m1, col, E), axis=1, keepdims=True)
    logits2 = jnp.where(col == a1, -jnp.inf, logits)
    m2 = jnp.max(logits2, axis=1, keepdims=True)
    a2 = jnp.min(jnp.where(logits2 == m2, col, E), axis=1, keepdims=True)
    score = (jnp.where(a1 == e, jax.nn.sigmoid(m1), 0.0)
             + jnp.where(a2 == e, jax.nn.sigmoid(m2), 0.0))

    xs = x * score
    gu = jnp.dot(xs, gu_ref[0], preferred_element_type=jnp.float32)
    g = gu[:, :FFN]
    u = gu[:, FFN:]
    h = u * (g * jax.nn.sigmoid(g))
    contrib = jnp.dot(h, dn_ref[0], preferred_element_type=jnp.float32)

    @pl.when(e == 0)
    def _():
        acc_ref[sl, :] = contrib

    @pl.when(e > 0)
    def _():
        acc_ref[sl, :] = acc_ref[sl, :] + contrib

    @pl.when(e == E - 1)
    def _():
        out_ref[...] = acc_ref[sl, :]


def _routed(hidden, router_w, gate_up_proj, down_proj):
    return pl.pallas_call(
        _routed_body,
        grid=(E, T // TM),
        in_specs=[
            pl.BlockSpec((T, H), lambda e, t: (0, 0)),
            pl.BlockSpec((E, H), lambda e, t: (0, 0)),
            pl.BlockSpec((1, H, 2 * FFN), lambda e, t: (e, 0, 0)),
            pl.BlockSpec((1, FFN, H), lambda e, t: (e, 0, 0)),
        ],
        out_specs=pl.BlockSpec((TM, H), lambda e, t: (t, 0)),
        out_shape=jax.ShapeDtypeStruct((T, H), jnp.float32),
        scratch_shapes=[pltpu.VMEM((T, H), jnp.float32)],
    )(hidden, router_w, gate_up_proj, down_proj)


def _shared_body(x_ref, shg_ref, shu_ref, shd_ref, part_ref, out_ref, acc_ref):
    j = pl.program_id(0)
    t = pl.program_id(1)
    sl = pl.ds(t * TM, TM)
    x = x_ref[sl, :]

    g = lax.dot_general(x, shg_ref[...], (((1,), (1,)), ((), ())),
                        preferred_element_type=jnp.float32)
    u = lax.dot_general(x, shu_ref[...], (((1,), (1,)), ((), ())),
                        preferred_element_type=jnp.float32)
    h = u * (g * jax.nn.sigmoid(g))
    contrib = lax.dot_general(h, shd_ref[...], (((1,), (1,)), ((), ())),
                              preferred_element_type=jnp.float32)

    @pl.when(j == 0)
    def _():
        acc_ref[sl, :] = part_ref[sl, :] + contrib

    @pl.when(j > 0)
    def _():
        acc_ref[sl, :] = acc_ref[sl, :] + contrib

    @pl.when(j == SFFN // FJ - 1)
    def _():
        out_ref[...] = acc_ref[sl, :]


def _shared(hidden, sh_gate_w, sh_up_w, sh_down_w, part):
    return pl.pallas_call(
        _shared_body,
        grid=(SFFN // FJ, T // TM),
        in_specs=[
            pl.BlockSpec((T, H), lambda j, t: (0, 0)),
            pl.BlockSpec((FJ, H), lambda j, t: (j, 0)),
            pl.BlockSpec((FJ, H), lambda j, t: (j, 0)),
            pl.BlockSpec((H, FJ), lambda j, t: (0, j)),
            pl.BlockSpec((T, H), lambda j, t: (0, 0)),
        ],
        out_specs=pl.BlockSpec((TM, H), lambda j, t: (t, 0)),
        out_shape=jax.ShapeDtypeStruct((T, H), jnp.float32),
        scratch_shapes=[pltpu.VMEM((T, H), jnp.float32)],
    )(hidden, sh_gate_w, sh_up_w, sh_down_w, part)


@jax.jit
def _moe(hidden, router_w, gate_up_proj, down_proj, sh_gate_w, sh_up_w, sh_down_w):
    part = _routed(hidden, router_w, gate_up_proj, down_proj)
    return _shared(hidden, sh_gate_w, sh_up_w, sh_down_w, part)


def kernel(hidden_states, router_w, gate_up_proj, down_proj, sh_gate_w, sh_up_w, sh_down_w):
    B, S, Hd = hidden_states.shape
    hidden = hidden_states.reshape(-1, Hd)
    out = _moe(hidden, router_w, gate_up_proj, down_proj, sh_gate_w, sh_up_w, sh_down_w)
    return out.reshape(B, S, Hd)


# speedup vs baseline: 1.6900x; 1.6900x over previous
import jax
import jax.numpy as jnp
from jax.experimental import pallas as pl

E = 8
TOP_K = 2
H = 1024
FFN = 512
SFFN = 2048
TM = 256


def _moe_body(x_ref, rw_ref, gu_ref, dn_ref, out_ref):
    x = x_ref[...]

    logits = jax.lax.dot_general(x, rw_ref[...], (((1,), (1,)), ((), ())),
                                 preferred_element_type=jnp.float32)
    col = jax.lax.broadcasted_iota(jnp.int32, (TM, E), 1)
    m1 = jnp.max(logits, axis=1, keepdims=True)
    a1 = jnp.min(jnp.where(logits == m1, col, E), axis=1, keepdims=True)
    logits2 = jnp.where(col == a1, -jnp.inf, logits)
    m2 = jnp.max(logits2, axis=1, keepdims=True)
    a2 = jnp.min(jnp.where(logits2 == m2, col, E), axis=1, keepdims=True)
    keep = (col == a1) | (col == a2)
    scores = jnp.where(keep, jax.nn.sigmoid(logits), 0.0)

    acc = jnp.zeros((TM, H), jnp.float32)
    for e in range(E):
        xs = x * scores[:, e:e + 1]
        gu = jnp.dot(xs, gu_ref[e], preferred_element_type=jnp.float32)
        g = gu[:, :FFN]
        u = gu[:, FFN:]
        h = u * (g * jax.nn.sigmoid(g))
        acc = acc + jnp.dot(h, dn_ref[e], preferred_element_type=jnp.float32)

    out_ref[...] = acc


def _shared_body(x_ref, shg_ref, shu_ref, shd_ref, part_ref, out_ref):
    x = x_ref[...]
    gsh = jax.lax.dot_general(x, shg_ref[...], (((1,), (1,)), ((), ())),
                              preferred_element_type=jnp.float32)
    ush = jax.lax.dot_general(x, shu_ref[...], (((1,), (1,)), ((), ())),
                              preferred_element_type=jnp.float32)
    hsh = ush * (gsh * jax.nn.sigmoid(gsh))
    out_ref[...] = part_ref[...] + jax.lax.dot_general(
        hsh, shd_ref[...], (((1,), (1,)), ((), ())),
        preferred_element_type=jnp.float32)


@jax.jit
def _moe(hidden, router_w, gate_up_proj, down_proj, sh_gate_w, sh_up_w, sh_down_w):
    T = hidden.shape[0]
    part = pl.pallas_call(
        _moe_body,
        grid=(T // TM,),
        in_specs=[
            pl.BlockSpec((TM, H), lambda t: (t, 0)),
            pl.BlockSpec((E, H), lambda t: (0, 0)),
            pl.BlockSpec((E, H, 2 * FFN), lambda t: (0, 0, 0)),
            pl.BlockSpec((E, FFN, H), lambda t: (0, 0, 0)),
        ],
        out_specs=pl.BlockSpec((TM, H), lambda t: (t, 0)),
        out_shape=jax.ShapeDtypeStruct((T, H), jnp.float32),
    )(hidden, router_w, gate_up_proj, down_proj)
    out = pl.pallas_call(
        _shared_body,
        grid=(T // TM,),
        in_specs=[
            pl.BlockSpec((TM, H), lambda t: (t, 0)),
            pl.BlockSpec((SFFN, H), lambda t: (0, 0)),
            pl.BlockSpec((SFFN, H), lambda t: (0, 0)),
            pl.BlockSpec((H, SFFN), lambda t: (0, 0)),
            pl.BlockSpec((TM, H), lambda t: (t, 0)),
        ],
        out_specs=pl.BlockSpec((TM, H), lambda t: (t, 0)),
        out_shape=jax.ShapeDtypeStruct((T, H), jnp.float32),
    )(hidden, sh_gate_w, sh_up_w, sh_down_w, part)
    return out


def kernel(hidden_states, router_w, gate_up_proj, down_proj, sh_gate_w, sh_up_w, sh_down_w):
    B, S, Hd = hidden_states.shape
    hidden = hidden_states.reshape(-1, Hd)
    out = _moe(hidden, router_w, gate_up_proj, down_proj, sh_gate_w, sh_up_w, sh_down_w)
    return out.reshape(B, S, Hd)
